# baseline (speedup 1.0000x reference)
import functools

import jax
import jax.numpy as jnp
from jax import lax
from jax.experimental import pallas as pl
from jax.experimental.pallas import tpu as pltpu
from jax.experimental.pallas import tpu_sc as plsc

_VOCAB = 1000000
_D = 16
_NUM_COLS = 5
_B = 16384
_NC, _NS = 2, 16
_NW = _NC * _NS
_BPW = _B // _NW
_CPW = _BPW * _NUM_COLS
_CHUNK = 128


def _sc_call(tgt_idx, ctx_idx, tgt_tab, ctx_tab):
    mesh = plsc.VectorSubcoreMesh(core_axis_name="c", subcore_axis_name="s")

    @functools.partial(
        pl.kernel,
        mesh=mesh,
        compiler_params=pltpu.CompilerParams(
            needs_layout_passes=False, use_tc_tiling_on_sc=False),
        out_type=jax.ShapeDtypeStruct((_B * _NUM_COLS,), jnp.float32),
        scratch_types=[
            pltpu.VMEM((_BPW,), jnp.int32),
            pltpu.VMEM((_CPW,), jnp.int32),
            pltpu.VMEM((_BPW, _D), jnp.float32),
            pltpu.VMEM((_CPW, _D), jnp.float32),
            pltpu.VMEM((_CPW,), jnp.float32),
            pltpu.SemaphoreType.DMA,
        ],
    )
    def body(tgt_idx_hbm, ctx_idx_hbm, tgt_tab_hbm, ctx_tab_hbm, out_hbm,
             tidx_v, cidx_v, trows_v, crows_v, out_v, sem):
        wid = lax.axis_index("s") * _NC + lax.axis_index("c")
        base = wid * _BPW
        cbase = wid * _CPW

        pltpu.sync_copy(tgt_idx_hbm.at[pl.ds(base, _BPW)], tidx_v)
        pltpu.sync_copy(ctx_idx_hbm.at[pl.ds(cbase, _CPW)], cidx_v)

        copies = []
        for j in range(_BPW // _CHUNK):
            s = pl.ds(j * _CHUNK, _CHUNK)
            copies.append(pltpu.async_copy(
                tgt_tab_hbm.at[tidx_v.at[s]], trows_v.at[s], sem))
        for j in range(_CPW // _CHUNK):
            s = pl.ds(j * _CHUNK, _CHUNK)
            copies.append(pltpu.async_copy(
                ctx_tab_hbm.at[cidx_v.at[s]], crows_v.at[s], sem))
        for c in copies:
            c.wait()

        lane = lax.iota(jnp.int32, 16)
        last = lane == 15

        def step(b, carry):
            tvec = trows_v[b]
            b5 = b * _NUM_COLS
            for c in range(_NUM_COLS):
                prod = crows_v[b5 + c] * tvec
                s = plsc.cumsum(prod)
                idx = jnp.zeros((16,), jnp.int32) + (b5 + c)
                plsc.store_scatter(out_v, [idx], s, mask=last)
            return carry

        lax.fori_loop(0, _BPW, step, 0)

        pltpu.sync_copy(out_v, out_hbm.at[pl.ds(cbase, _CPW)])

    return body(tgt_idx, ctx_idx, tgt_tab, ctx_tab)


def kernel(target, context, target_table, context_table):
    out = _sc_call(target.reshape(-1), context.reshape(-1),
                   target_table, context_table)
    return out.reshape(_B, _NUM_COLS)


# speedup vs baseline: 1.0074x; 1.0074x over previous
import functools

import jax
import jax.numpy as jnp
from jax import lax
from jax.experimental import pallas as pl
from jax.experimental.pallas import tpu as pltpu
from jax.experimental.pallas import tpu_sc as plsc

_VOCAB = 1000000
_D = 16
_NUM_COLS = 5
_B = 16384
_NC, _NS = 2, 16
_NW = _NC * _NS
_BPW = _B // _NW
_CPW = _BPW * _NUM_COLS
_CHUNK = 128
_NBLK = _CPW // _D


def _sc_call(tgt_idx, ctx_idx, tgt_tab, ctx_tab):
    mesh = plsc.VectorSubcoreMesh(core_axis_name="c", subcore_axis_name="s")

    @functools.partial(
        pl.kernel,
        mesh=mesh,
        compiler_params=pltpu.CompilerParams(
            needs_layout_passes=False, use_tc_tiling_on_sc=False),
        out_type=jax.ShapeDtypeStruct((_B * _NUM_COLS,), jnp.float32),
        scratch_types=[
            pltpu.VMEM((_BPW,), jnp.int32),
            pltpu.VMEM((_CPW,), jnp.int32),
            pltpu.VMEM((_BPW, _D), jnp.float32),
            pltpu.VMEM((_CPW, _D), jnp.float32),
            pltpu.VMEM((_CPW,), jnp.float32),
            pltpu.SemaphoreType.DMA,
        ],
    )
    def body(tgt_idx_hbm, ctx_idx_hbm, tgt_tab_hbm, ctx_tab_hbm, out_hbm,
             tidx_v, cidx_v, trows_v, crows_v, out_v, sem):
        wid = lax.axis_index("s") * _NC + lax.axis_index("c")
        base = wid * _BPW
        cbase = wid * _CPW

        pltpu.sync_copy(tgt_idx_hbm.at[pl.ds(base, _BPW)], tidx_v)
        pltpu.sync_copy(ctx_idx_hbm.at[pl.ds(cbase, _CPW)], cidx_v)

        copies = []
        for j in range(_BPW // _CHUNK):
            s = pl.ds(j * _CHUNK, _CHUNK)
            copies.append(pltpu.async_copy(
                tgt_tab_hbm.at[tidx_v.at[s]], trows_v.at[s], sem))
        for j in range(_CPW // _CHUNK):
            s = pl.ds(j * _CHUNK, _CHUNK)
            copies.append(pltpu.async_copy(
                ctx_tab_hbm.at[cidx_v.at[s]], crows_v.at[s], sem))
        for c in copies:
            c.wait()

        lane = lax.iota(jnp.int32, 16)

        def step(k, carry):
            row = k * _D + lane
            trow = lax.shift_right_logical(row * 13108, 16)
            acc = jnp.zeros((16,), jnp.float32)
            for f in range(_D):
                fvec = jnp.zeros((16,), jnp.int32) + f
                cv = plsc.load_gather(crows_v, [row, fvec])
                tv = plsc.load_gather(trows_v, [trow, fvec])
                acc = acc + cv * tv
            out_v[pl.ds(k * _D, _D)] = acc
            return carry

        lax.fori_loop(0, _NBLK, step, 0)

        pltpu.sync_copy(out_v, out_hbm.at[pl.ds(cbase, _CPW)])

    return body(tgt_idx, ctx_idx, tgt_tab, ctx_tab)


def kernel(target, context, target_table, context_table):
    out = _sc_call(target.reshape(-1), context.reshape(-1),
                   target_table, context_table)
    return out.reshape(_B, _NUM_COLS)


# speedup vs baseline: 1.0086x; 1.0012x over previous
import functools

import jax
import jax.numpy as jnp
from jax import lax
from jax.experimental import pallas as pl
from jax.experimental.pallas import tpu as pltpu
from jax.experimental.pallas import tpu_sc as plsc

_VOCAB = 1000000
_D = 16
_NUM_COLS = 5
_B = 16384
_NC, _NS = 2, 16
_NW = _NC * _NS
_BPW = _B // _NW
_CPW = _BPW * _NUM_COLS
_TR = _BPW // 128
_CR = _CPW // 128
_NBLK = _CPW // _D


def _sc_call(tgt_idx2, ctx_idx2, tgt_tab, ctx_tab):
    mesh = plsc.VectorSubcoreMesh(core_axis_name="c", subcore_axis_name="s")

    @functools.partial(
        pl.kernel,
        mesh=mesh,
        compiler_params=pltpu.CompilerParams(
            needs_layout_passes=False, use_tc_tiling_on_sc=False),
        out_type=jax.ShapeDtypeStruct((_B * _NUM_COLS,), jnp.float32),
        scratch_types=[
            pltpu.VMEM((_TR, 128), jnp.int32),
            pltpu.VMEM((_CR, 128), jnp.int32),
            pltpu.VMEM((_TR, 128, _D), jnp.float32),
            pltpu.VMEM((_CR, 128, _D), jnp.float32),
            pltpu.VMEM((_CPW,), jnp.float32),
            pltpu.SemaphoreType.DMA,
        ],
    )
    def body(tgt_idx_hbm, ctx_idx_hbm, tgt_tab_hbm, ctx_tab_hbm, out_hbm,
             tidx_v, cidx_v, trows_v, crows_v, out_v, sem):
        wid = lax.axis_index("s") * _NC + lax.axis_index("c")
        cbase = wid * _CPW

        pltpu.sync_copy(tgt_idx_hbm.at[pl.ds(wid * _TR, _TR)], tidx_v)
        pltpu.sync_copy(ctx_idx_hbm.at[pl.ds(wid * _CR, _CR)], cidx_v)

        copies = []
        for r in range(_TR):
            copies.append(pltpu.async_copy(
                tgt_tab_hbm.at[tidx_v.at[r]], trows_v.at[r], sem))
        for r in range(_CR):
            copies.append(pltpu.async_copy(
                ctx_tab_hbm.at[cidx_v.at[r]], crows_v.at[r], sem))
        for c in copies:
            c.wait()

        lane = lax.iota(jnp.int32, 16)
        m127 = jnp.zeros((16,), jnp.int32) + 127

        def step(k, carry):
            i = k * _D + lane
            t = lax.shift_right_logical(i * 13108, 16)
            i0 = lax.shift_right_logical(i, 7)
            i1 = lax.bitwise_and(i, m127)
            t0 = lax.shift_right_logical(t, 7)
            t1 = lax.bitwise_and(t, m127)
            acc = jnp.zeros((16,), jnp.float32)
            for f in range(_D):
                fvec = jnp.zeros((16,), jnp.int32) + f
                cv = plsc.load_gather(crows_v, [i0, i1, fvec])
                tv = plsc.load_gather(trows_v, [t0, t1, fvec])
                acc = acc + cv * tv
            out_v[pl.ds(k * _D, _D)] = acc
            return carry

        lax.fori_loop(0, _NBLK, step, 0)

        pltpu.sync_copy(out_v, out_hbm.at[pl.ds(cbase, _CPW)])

    return body(tgt_idx2, ctx_idx2, tgt_tab, ctx_tab)


def kernel(target, context, target_table, context_table):
    out = _sc_call(target.reshape(-1).reshape(_NW * _TR, 128),
                   context.reshape(-1).reshape(_NW * _CR, 128),
                   target_table, context_table)
    return out.reshape(_B, _NUM_COLS)
